# Initial kernel scaffold; baseline (speedup 1.0000x reference)
import jax
import jax.numpy as jnp
from jax.experimental import pallas as pl


def kernel(inputs, weight):
    raise NotImplementedError("write your pallas kernel here")


# speedup vs baseline: 8.1493x; 8.1493x over previous
import functools

import jax
import jax.numpy as jnp
from jax import lax
from jax.experimental import pallas as pl
from jax.experimental.pallas import tpu as pltpu
from jax.experimental.pallas import tpu_sc as plsc

VOCAB = 100000
D = 128
NC = 2
NS = 16
NW = NC * NS


def _embed_lookup(idx_flat, weight, *, n_rows, chunk):
    b_per_w = n_rows // NW
    n_chunks = b_per_w // chunk
    mesh = plsc.VectorSubcoreMesh(core_axis_name="c", subcore_axis_name="s")

    @functools.partial(
        pl.kernel,
        mesh=mesh,
        out_type=jax.ShapeDtypeStruct((n_rows, D), jnp.float32),
        scratch_types=[
            pltpu.VMEM((chunk,), jnp.int32),
            pltpu.VMEM((chunk, D), jnp.float32),
            pltpu.SemaphoreType.DMA,
        ],
    )
    def k(idx_hbm, table_hbm, out_hbm, idx_v, rows_v, sem):
        wid = lax.axis_index("s") * NC + lax.axis_index("c")
        base = wid * b_per_w

        @pl.loop(0, n_chunks)
        def _(ci):
            off = base + ci * chunk
            pltpu.sync_copy(idx_hbm.at[pl.ds(off, chunk)], idx_v)
            pltpu.async_copy(table_hbm.at[idx_v], rows_v, sem).wait()
            pltpu.sync_copy(rows_v, out_hbm.at[pl.ds(off, chunk)])

    return k(idx_flat, weight)


def kernel(inputs, weight):
    b, t = inputs.shape
    n_rows = b * t
    idx_flat = inputs.reshape(n_rows).astype(jnp.int32)
    out = _embed_lookup(idx_flat, weight, n_rows=n_rows, chunk=512)
    return out.reshape(b, t, D)


# speedup vs baseline: 9.2644x; 1.1368x over previous
import functools

import jax
import jax.numpy as jnp
from jax import lax
from jax.experimental import pallas as pl
from jax.experimental.pallas import tpu as pltpu
from jax.experimental.pallas import tpu_sc as plsc

VOCAB = 100000
D = 128
NC = 2
NS = 16
NW = NC * NS


def _embed_lookup(idx_flat, weight, *, n_rows, chunk):
    b_per_w = n_rows // NW
    n_chunks = b_per_w // chunk
    assert n_chunks % 2 == 0 and n_chunks >= 4
    mesh = plsc.VectorSubcoreMesh(core_axis_name="c", subcore_axis_name="s")

    @functools.partial(
        pl.kernel,
        mesh=mesh,
        out_type=jax.ShapeDtypeStruct((n_rows, D), jnp.float32),
        scratch_types=[
            pltpu.VMEM((chunk,), jnp.int32),
            pltpu.VMEM((chunk,), jnp.int32),
            pltpu.VMEM((chunk, D), jnp.float32),
            pltpu.VMEM((chunk, D), jnp.float32),
            pltpu.SemaphoreType.DMA,
            pltpu.SemaphoreType.DMA,
            pltpu.SemaphoreType.DMA,
            pltpu.SemaphoreType.DMA,
        ],
    )
    def k(idx_hbm, table_hbm, out_hbm, i0, i1, r0, r1, g0, g1, s0, s1):
        wid = lax.axis_index("s") * NC + lax.axis_index("c")
        base = wid * b_per_w
        idx_v = (i0, i1)
        rows_v = (r0, r1)
        gsem = (g0, g1)
        ssem = (s0, s1)

        def gather_start(ci, b):
            off = base + ci * chunk
            pltpu.sync_copy(idx_hbm.at[pl.ds(off, chunk)], idx_v[b])
            pltpu.async_copy(table_hbm.at[idx_v[b]], rows_v[b], gsem[b])

        def gather_wait(b):
            pltpu.make_async_copy(table_hbm.at[idx_v[b]], rows_v[b],
                                  gsem[b]).wait()

        def store_start(ci, b):
            off = base + ci * chunk
            pltpu.async_copy(rows_v[b], out_hbm.at[pl.ds(off, chunk)],
                             ssem[b])

        def store_wait(ci, b):
            off = base + ci * chunk
            pltpu.make_async_copy(rows_v[b], out_hbm.at[pl.ds(off, chunk)],
                                  ssem[b]).wait()

        gather_start(0, 0)
        gather_start(1, 1)

        @pl.loop(0, n_chunks - 2, step=2)
        def _(ci):
            for ph in range(2):
                c = ci + ph
                gather_wait(ph)
                store_start(c, ph)
                store_wait(c, ph)
                gather_start(c + 2, ph)

        for ph in range(2):
            c = n_chunks - 2 + ph
            gather_wait(ph)
            store_start(c, ph)
        for ph in range(2):
            store_wait(n_chunks - 2 + ph, ph)

    return k(idx_flat, weight)


def kernel(inputs, weight):
    b, t = inputs.shape
    n_rows = b * t
    idx_flat = inputs.reshape(n_rows).astype(jnp.int32)
    out = _embed_lookup(idx_flat, weight, n_rows=n_rows, chunk=400)
    return out.reshape(b, t, D)
